# baseline (speedup 1.0000x reference)
import functools

import jax
import jax.numpy as jnp
from jax import lax
from jax.experimental import pallas as pl
from jax.experimental.pallas import tpu as pltpu
from jax.experimental.pallas import tpu_sc as plsc

HIDDEN = 2048
NUM_EXPERTS = 64
TOP_K = 8
TOKENS = 16384

MM_BLK = 1024
NC, NS, L = 2, 16, 16
NW = NC * NS
N_CHUNKS = 4
CHUNK = TOKENS // N_CHUNKS


def _matmul_block(x_ref, w_ref, logits_ref):
    logits_ref[...] = jax.lax.dot_general(
        x_ref[...], w_ref[...], dimension_numbers=(((1,), (1,)), ((), ())),
        preferred_element_type=jnp.float32)


def _tc_logits_chunk(hidden_states, gate_weight, chunk):
    base = chunk * (CHUNK // MM_BLK)
    return pl.pallas_call(
        _matmul_block,
        grid=(CHUNK // MM_BLK,),
        in_specs=[
            pl.BlockSpec((MM_BLK, HIDDEN), lambda i: (base + i, 0)),
            pl.BlockSpec((NUM_EXPERTS, HIDDEN), lambda i: (0, 0)),
        ],
        out_specs=pl.BlockSpec((MM_BLK, NUM_EXPERTS), lambda i: (i, 0)),
        out_shape=jax.ShapeDtypeStruct((CHUNK, NUM_EXPERTS), jnp.float32),
        compiler_params=pltpu.CompilerParams(
            dimension_semantics=("arbitrary",)),
    )(hidden_states, gate_weight)


def _merge_desc(ak, av, bk, bv):
    rbk = lax.rev(bk, (0,))
    rbv = lax.rev(bv, (0,))
    take_a = (ak > rbk) | ((ak == rbk) & (av < rbv))
    mk = jnp.where(take_a, ak, rbk)
    mv = jnp.where(take_a, av, rbv)
    return plsc.sort_key_val(mk, mv, descending=True)


def _make_sc_route(rows):
    r_per_w = rows // NW
    mesh = plsc.VectorSubcoreMesh(core_axis_name="c", subcore_axis_name="s")

    @functools.partial(
        pl.kernel,
        mesh=mesh,
        out_type=[
            jax.ShapeDtypeStruct((rows * TOP_K,), jnp.int32),
            jax.ShapeDtypeStruct((rows * TOP_K,), jnp.float32),
        ],
        scratch_types=[
            pltpu.VMEM((r_per_w, NUM_EXPERTS), jnp.float32),
            pltpu.VMEM((r_per_w * TOP_K,), jnp.int32),
            pltpu.VMEM((r_per_w * TOP_K,), jnp.float32),
        ],
        compiler_params=pltpu.CompilerParams(needs_layout_passes=False),
    )
    def sc_topk(logits_hbm, ids_hbm, wts_hbm, lg_v, ids_v, wts_v):
        wid = lax.axis_index("s") * NC + lax.axis_index("c")
        base = wid * r_per_w
        pltpu.sync_copy(logits_hbm.at[pl.ds(base, r_per_w)], lg_v)

        lane = lax.iota(jnp.int32, L)
        lane8 = lane < TOP_K

        @plsc.parallel_loop(0, r_per_w, unroll=8)
        def row_body(r):
            sorted_kv = []
            for c in range(NUM_EXPERTS // L):
                k = lg_v[r, pl.ds(c * L, L)]
                sorted_kv.append(
                    plsc.sort_key_val(k, lane + c * L, descending=True))
            t01 = _merge_desc(*sorted_kv[0], *sorted_kv[1])
            t23 = _merge_desc(*sorted_kv[2], *sorted_kv[3])
            fk, fv = _merge_desc(*t01, *t23)
            e = jnp.exp(fk - jnp.max(fk))
            e8 = jnp.where(lane8, e, 0.0)
            w = e8 / jnp.sum(e8)
            pos = r * TOP_K + lane
            plsc.store_scatter(ids_v, [pos], fv, mask=lane8)
            plsc.store_scatter(wts_v, [pos], w, mask=lane8)

        pltpu.sync_copy(ids_v, ids_hbm.at[pl.ds(base * TOP_K, r_per_w * TOP_K)])
        pltpu.sync_copy(wts_v, wts_hbm.at[pl.ds(base * TOP_K, r_per_w * TOP_K)])

    return sc_topk


_sc_route_chunk = _make_sc_route(CHUNK)


@jax.jit
def kernel(hidden_states, gate_weight):
    lg_chunks, id_chunks, wt_chunks = [], [], []
    for c in range(N_CHUNKS):
        lg = _tc_logits_chunk(hidden_states, gate_weight, c)
        ids_flat, wts_flat = _sc_route_chunk(lg)
        lg_chunks.append(lg)
        id_chunks.append(ids_flat)
        wt_chunks.append(wts_flat)
    logits = jnp.concatenate(lg_chunks, axis=0)
    ids = jnp.concatenate(id_chunks, axis=0).reshape(TOKENS, TOP_K)
    wts = jnp.concatenate(wt_chunks, axis=0).reshape(TOKENS, TOP_K)
    return ids, wts, logits


# speedup vs baseline: 1.3171x; 1.3171x over previous
import functools

import jax
import jax.numpy as jnp
from jax import lax
from jax.experimental import pallas as pl
from jax.experimental.pallas import tpu as pltpu
from jax.experimental.pallas import tpu_sc as plsc

HIDDEN = 2048
NUM_EXPERTS = 64
TOP_K = 8
TOKENS = 16384

MM_BLK = 1024
NC, NS, L = 2, 16, 16
NW = NC * NS
N_CHUNKS = 4
CHUNK = TOKENS // N_CHUNKS


def _matmul_block(x_ref, w_ref, logits_ref):
    logits_ref[...] = jax.lax.dot_general(
        w_ref[...], x_ref[...], dimension_numbers=(((1,), (1,)), ((), ())),
        preferred_element_type=jnp.float32)


def _tc_logits_t_chunk(hidden_states, gate_weight, chunk):
    base = chunk * (CHUNK // MM_BLK)
    return pl.pallas_call(
        _matmul_block,
        grid=(CHUNK // MM_BLK,),
        in_specs=[
            pl.BlockSpec((MM_BLK, HIDDEN), lambda i: (base + i, 0)),
            pl.BlockSpec((NUM_EXPERTS, HIDDEN), lambda i: (0, 0)),
        ],
        out_specs=pl.BlockSpec((NUM_EXPERTS, MM_BLK), lambda i: (0, i)),
        out_shape=jax.ShapeDtypeStruct((NUM_EXPERTS, CHUNK), jnp.float32),
        compiler_params=pltpu.CompilerParams(
            dimension_semantics=("arbitrary",)),
    )(hidden_states, gate_weight)


def _merge_desc(ak, av, bk, bv):
    rbk = lax.rev(bk, (0,))
    rbv = lax.rev(bv, (0,))
    take_a = (ak > rbk) | ((ak == rbk) & (av < rbv))
    mk = jnp.where(take_a, ak, rbk)
    mv = jnp.where(take_a, av, rbv)
    return plsc.sort_key_val(mk, mv, descending=True)


def _make_sc_route(cols):
    c_per_w = cols // NW
    mesh = plsc.VectorSubcoreMesh(core_axis_name="c", subcore_axis_name="s")

    @functools.partial(
        pl.kernel,
        mesh=mesh,
        out_type=[
            jax.ShapeDtypeStruct((TOP_K, cols), jnp.int32),
            jax.ShapeDtypeStruct((TOP_K, cols), jnp.float32),
        ],
        scratch_types=[
            pltpu.VMEM((NUM_EXPERTS, cols // NW), jnp.float32),
            pltpu.VMEM((TOP_K, cols // NW), jnp.int32),
            pltpu.VMEM((TOP_K, cols // NW), jnp.float32),
        ],
        compiler_params=pltpu.CompilerParams(needs_layout_passes=False),
    )
    def sc_topk(logits_hbm, ids_hbm, wts_hbm, lg_v, ids_v, wts_v):
        wid = lax.axis_index("s") * NC + lax.axis_index("c")
        base = wid * c_per_w
        pltpu.sync_copy(logits_hbm.at[:, pl.ds(base, c_per_w)], lg_v)

        lane = lax.iota(jnp.int32, L)
        lane8 = lane < TOP_K

        @plsc.parallel_loop(0, c_per_w, unroll=8)
        def tok_body(t):
            tcol = jnp.full((L,), 0, jnp.int32) + t
            sorted_kv = []
            for c in range(NUM_EXPERTS // L):
                k = plsc.load_gather(lg_v, [lane + c * L, tcol])
                sorted_kv.append(
                    plsc.sort_key_val(k, lane + c * L, descending=True))
            t01 = _merge_desc(*sorted_kv[0], *sorted_kv[1])
            t23 = _merge_desc(*sorted_kv[2], *sorted_kv[3])
            fk, fv = _merge_desc(*t01, *t23)
            e = jnp.exp(fk - jnp.max(fk))
            e8 = jnp.where(lane8, e, 0.0)
            w = e8 / jnp.sum(e8)
            plsc.store_scatter(ids_v, [lane, tcol], fv, mask=lane8)
            plsc.store_scatter(wts_v, [lane, tcol], w, mask=lane8)

        pltpu.sync_copy(ids_v, ids_hbm.at[:, pl.ds(base, c_per_w)])
        pltpu.sync_copy(wts_v, wts_hbm.at[:, pl.ds(base, c_per_w)])

    return sc_topk


_sc_route_chunk = _make_sc_route(CHUNK)


@jax.jit
def kernel(hidden_states, gate_weight):
    lg_chunks, id_chunks, wt_chunks = [], [], []
    for c in range(N_CHUNKS):
        lg_t = _tc_logits_t_chunk(hidden_states, gate_weight, c)
        ids_t, wts_t = _sc_route_chunk(lg_t)
        lg_chunks.append(lg_t)
        id_chunks.append(ids_t)
        wt_chunks.append(wts_t)
    logits = jnp.concatenate(lg_chunks, axis=1).T
    ids = jnp.concatenate(id_chunks, axis=1).T
    wts = jnp.concatenate(wt_chunks, axis=1).T
    return ids, wts, logits
